# baseline (speedup 1.0000x reference)
import functools

import jax
import jax.numpy as jnp
from jax import lax
from jax.experimental import pallas as pl
from jax.experimental.pallas import tpu as pltpu
from jax.experimental.pallas import tpu_sc as plsc

D = 64
D3 = 3 * D
B = 16384
REG = 0.01

NC = 2
NS = 16
NW = NC * NS
BPW = B // NW
CHUNK = 128
NCHUNK = BPW // CHUNK


def _gather_body(idx_p, idx_q, idx_r, tab_p, tab_q, tab_r,
                 out_p, out_q, out_r, idx_v, rows_v, sem):
    wid = lax.axis_index("s") * NC + lax.axis_index("c")
    base_i = wid * NCHUNK
    base_o = wid * BPW
    for idx_hbm, table, out in ((idx_p, tab_p, out_p),
                                (idx_q, tab_q, out_q),
                                (idx_r, tab_r, out_r)):
        pltpu.sync_copy(idx_hbm.at[pl.ds(base_i, NCHUNK)], idx_v)
        copies = [
            pltpu.async_copy(table.at[idx_v.at[j]],
                             rows_v.at[pl.ds(j * CHUNK, CHUNK)], sem)
            for j in range(NCHUNK)
        ]
        for c in copies:
            c.wait()
        pltpu.sync_copy(rows_v, out.at[pl.ds(base_o, BPW)])


@functools.cache
def _sc_gather():
    return pl.kernel(
        _gather_body,
        out_type=[jax.ShapeDtypeStruct((B, D), jnp.float32)] * 3,
        mesh=plsc.VectorSubcoreMesh(core_axis_name="c", subcore_axis_name="s"),
        scratch_types=[
            pltpu.VMEM((NCHUNK, CHUNK), jnp.int32),
            pltpu.VMEM((BPW, D), jnp.float32),
            pltpu.SemaphoreType.DMA,
        ],
        compiler_params=pltpu.CompilerParams(use_tc_tiling_on_sc=False),
    )


def _mlp_body(xp_ref, xq_ref, xr_ref, w1_ref, b1_ref, w2_ref, b2_ref,
              inf_ref, regs_ref):
    w1 = w1_ref[...]
    n1 = jnp.sqrt(jnp.sum(w1 * w1, axis=1, keepdims=True))
    w1c = w1 / jnp.maximum(n1, 1.0)
    w2 = w2_ref[...]
    n2 = jnp.sqrt(jnp.sum(w2 * w2, axis=1, keepdims=True))
    w2c = w2 / jnp.maximum(n2, 1.0)

    xp = xp_ref[...]
    xq = xq_ref[...]
    xr = xr_ref[...]

    dn = (((1,), (1,)), ((), ()))
    acc = lax.dot_general(w1c[:, :D], xp, dn,
                          precision=lax.Precision.HIGHEST,
                          preferred_element_type=jnp.float32)
    acc = acc + lax.dot_general(w1c[:, D:2 * D], xq, dn,
                                precision=lax.Precision.HIGHEST,
                                preferred_element_type=jnp.float32)
    acc = acc + lax.dot_general(w1c[:, 2 * D:], xr, dn,
                                precision=lax.Precision.HIGHEST,
                                preferred_element_type=jnp.float32)
    h = 1.0 / (1.0 + jnp.exp(-(acc + b1_ref[...])))
    inf = lax.dot_general(w2c, h, (((1,), (0,)), ((), ())),
                          precision=lax.Precision.HIGHEST,
                          preferred_element_type=jnp.float32)
    inf_ref[...] = inf + b2_ref[...]

    regs = REG * (jnp.sqrt(jnp.sum(xp * xp)) +
                  jnp.sqrt(jnp.sum(xq * xq)) +
                  jnp.sqrt(jnp.sum(xr * xr)))
    regs_ref[...] = jnp.broadcast_to(regs, (1, 1))


def _mlp(xp, xq, xr, W1, b1_col, W2, b2_2d):
    return pl.pallas_call(
        _mlp_body,
        out_shape=(jax.ShapeDtypeStruct((1, B), jnp.float32),
                   jax.ShapeDtypeStruct((1, 1), jnp.float32)),
    )(xp, xq, xr, W1, b1_col, W2, b2_2d)


def kernel(ps, qs, rs, table_p, table_q, table_r, W1, b1, W2, b2):
    ip = ps.astype(jnp.int32).reshape(NW * NCHUNK, CHUNK)
    iq = qs.astype(jnp.int32).reshape(NW * NCHUNK, CHUNK)
    ir = rs.astype(jnp.int32).reshape(NW * NCHUNK, CHUNK)
    xp, xq, xr = _sc_gather()(ip, iq, ir, table_p, table_q, table_r)
    inf, regs = _mlp(xp, xq, xr, W1, b1.reshape(D3, 1), W2, b2.reshape(1, 1))
    return inf.reshape(B, 1), regs[0, 0]


# speedup vs baseline: 1.5160x; 1.5160x over previous
import functools

import jax
import jax.numpy as jnp
from jax import lax
from jax.experimental import pallas as pl
from jax.experimental.pallas import tpu as pltpu
from jax.experimental.pallas import tpu_sc as plsc

D = 64
D3 = 3 * D
B = 16384
REG = 0.01

NC = 2
NS = 16
NW = NC * NS
BPW = B // NW
CHUNK = 128
NCHUNK = BPW // CHUNK


def _gather_body(idx_p, idx_q, idx_r, tab_p, tab_q, tab_r,
                 out_p, out_q, out_r, idx_v, rows_v, sem):
    wid = lax.axis_index("s") * NC + lax.axis_index("c")
    base = wid * BPW
    for idx_hbm, table, out in ((idx_p, tab_p, out_p),
                                (idx_q, tab_q, out_q),
                                (idx_r, tab_r, out_r)):
        pltpu.sync_copy(idx_hbm.at[pl.ds(base, BPW)], idx_v)

        def fire(g, _, table=table):
            v = idx_v[pl.ds(g * 16, 16)]
            for j in range(16):
                row = v[j]
                pltpu.make_async_copy(table.at[pl.ds(row, 1)],
                                      rows_v.at[pl.ds(g * 16 + j, 1)],
                                      sem).start()
            return 0

        lax.fori_loop(0, BPW // 16, fire, 0)
        pltpu.make_async_copy(table.at[pl.ds(0, BPW)], rows_v, sem).wait()
        pltpu.sync_copy(rows_v, out.at[pl.ds(base, BPW)])


@functools.cache
def _sc_gather():
    return pl.kernel(
        _gather_body,
        out_type=[jax.ShapeDtypeStruct((B, D), jnp.float32)] * 3,
        mesh=plsc.VectorSubcoreMesh(core_axis_name="c", subcore_axis_name="s"),
        scratch_types=[
            pltpu.VMEM((BPW,), jnp.int32),
            pltpu.VMEM((BPW, D), jnp.float32),
            pltpu.SemaphoreType.DMA,
        ],
    )


def _mlp_body(xp_ref, xq_ref, xr_ref, w1_ref, b1_ref, w2_ref, b2_ref,
              inf_ref, regs_ref):
    w1 = w1_ref[...]
    n1 = jnp.sqrt(jnp.sum(w1 * w1, axis=1, keepdims=True))
    w1c = w1 / jnp.maximum(n1, 1.0)
    w2 = w2_ref[...]
    n2 = jnp.sqrt(jnp.sum(w2 * w2, axis=1, keepdims=True))
    w2c = w2 / jnp.maximum(n2, 1.0)

    xp = xp_ref[...]
    xq = xq_ref[...]
    xr = xr_ref[...]

    dn = (((1,), (1,)), ((), ()))
    acc = lax.dot_general(w1c[:, :D], xp, dn,
                          precision=lax.Precision.HIGHEST,
                          preferred_element_type=jnp.float32)
    acc = acc + lax.dot_general(w1c[:, D:2 * D], xq, dn,
                                precision=lax.Precision.HIGHEST,
                                preferred_element_type=jnp.float32)
    acc = acc + lax.dot_general(w1c[:, 2 * D:], xr, dn,
                                precision=lax.Precision.HIGHEST,
                                preferred_element_type=jnp.float32)
    h = 1.0 / (1.0 + jnp.exp(-(acc + b1_ref[...])))
    inf = lax.dot_general(w2c, h, (((1,), (0,)), ((), ())),
                          precision=lax.Precision.HIGHEST,
                          preferred_element_type=jnp.float32)
    inf_ref[...] = inf + b2_ref[...]

    regs = REG * (jnp.sqrt(jnp.sum(xp * xp)) +
                  jnp.sqrt(jnp.sum(xq * xq)) +
                  jnp.sqrt(jnp.sum(xr * xr)))
    regs_ref[...] = jnp.broadcast_to(regs, (1, 1))


def _mlp(xp, xq, xr, W1, b1_col, W2, b2_2d):
    return pl.pallas_call(
        _mlp_body,
        out_shape=(jax.ShapeDtypeStruct((1, B), jnp.float32),
                   jax.ShapeDtypeStruct((1, 1), jnp.float32)),
    )(xp, xq, xr, W1, b1_col, W2, b2_2d)


def kernel(ps, qs, rs, table_p, table_q, table_r, W1, b1, W2, b2):
    ip = ps.astype(jnp.int32)
    iq = qs.astype(jnp.int32)
    ir = rs.astype(jnp.int32)
    xp, xq, xr = _sc_gather()(ip, iq, ir, table_p, table_q, table_r)
    inf, regs = _mlp(xp, xq, xr, W1, b1.reshape(D3, 1), W2, b2.reshape(1, 1))
    return inf.reshape(B, 1), regs[0, 0]
